# Initial kernel scaffold; baseline (speedup 1.0000x reference)
import jax
import jax.numpy as jnp
from jax.experimental import pallas as pl


def kernel(inputs):
    raise NotImplementedError("write your pallas kernel here")


# speedup vs baseline: 3.0523x; 3.0523x over previous
import math

import jax
import jax.numpy as jnp
import numpy as np
from jax.experimental import pallas as pl
from jax.experimental.pallas import tpu as pltpu

_D = 768
_W = 256
_S = 32
_CAT = 3 * _W


def _make_pe_rows() -> jnp.ndarray:
    pe = np.zeros((_CAT, _D), dtype=np.float32)
    position = np.arange(0, _CAT, dtype=np.float32)[:, None]
    div_term = np.exp(
        np.arange(0, _D, 2, dtype=np.float32) * (-math.log(10000.0) / _D)
    )
    pe[:, 0::2] = np.sin(position * div_term)
    pe[:, 1::2] = np.cos(position * div_term)
    return jnp.asarray(pe)


def _top3_kernel(first_ref, top3_ref):
    f = first_ref[...]
    soft = jnp.sum(jnp.abs(f[:, None, :] - f[None, :, :]), axis=-1)
    col = jax.lax.broadcasted_iota(jnp.int32, (_S, _S), 1)
    cols = []
    for _ in range(3):
        m = jnp.min(soft, axis=1, keepdims=True)
        cand = jnp.where(soft == m, col, _S)
        j = jnp.min(cand, axis=1, keepdims=True)
        cols.append(j)
        soft = jnp.where(col == j, jnp.float32(jnp.inf), soft)
    top3_ref[...] = jnp.concatenate(cols, axis=1).astype(jnp.int32)


def _attn_kernel(top3_ref, a_ref, b_ref, c_ref, pe_ref, out_ref):
    del top3_ref
    x = jnp.concatenate([a_ref[0], b_ref[0], c_ref[0]], axis=0) + pe_ref[...]
    q = x[:_W]
    scores = jax.lax.dot_general(
        q, x, (((1,), (1,)), ((), ())), preferred_element_type=jnp.float32
    ) * (1.0 / math.sqrt(_D))
    p = jax.nn.softmax(scores, axis=-1)
    out_ref[0] = jnp.dot(p, x, preferred_element_type=jnp.float32)


def kernel(inputs):
    first = inputs[:, 0, :]
    top3 = pl.pallas_call(
        _top3_kernel,
        out_shape=jax.ShapeDtypeStruct((_S, 3), jnp.int32),
    )(first)

    pe = _make_pe_rows()
    grid_spec = pltpu.PrefetchScalarGridSpec(
        num_scalar_prefetch=1,
        grid=(_S,),
        in_specs=[
            pl.BlockSpec((1, _W, _D), lambda i, t: (t[i, 0], 0, 0)),
            pl.BlockSpec((1, _W, _D), lambda i, t: (t[i, 1], 0, 0)),
            pl.BlockSpec((1, _W, _D), lambda i, t: (t[i, 2], 0, 0)),
            pl.BlockSpec((_CAT, _D), lambda i, t: (0, 0)),
        ],
        out_specs=pl.BlockSpec((1, _W, _D), lambda i, t: (i, 0, 0)),
    )
    fused = pl.pallas_call(
        _attn_kernel,
        grid_spec=grid_spec,
        out_shape=jax.ShapeDtypeStruct((_S, _W, _D), jnp.float32),
    )(top3, inputs, inputs, inputs, pe)
    return fused[:, : _W - 1, :]
